# baseline (speedup 1.0000x reference)
import jax
import jax.numpy as jnp
from jax.experimental import pallas as pl
from jax.experimental.pallas import tpu as pltpu

_NUM_IMAGES = 4
_NANCH = 20000
_NANCH_PAD = 20480
_PRE = 2000
_PAD = 2048
_POST = 1000
_B = 128
_NBLK = _PAD // _B
_IMG_H = 800.0
_IMG_W = 800.0
_NMS_THRESH = 0.7
_SCORE_THRESH = 0.0
_MIN_SIZE = 0.001


def _cutoff_body(obj_ref, sel_ref):
    obj = obj_ref[...] + 0.0
    bits = jax.lax.bitcast_convert_type(obj, jnp.int32)
    key = jnp.where(bits >= 0, bits, jnp.int32(-2147483648) - bits)
    col = jax.lax.broadcasted_iota(jnp.int32, (1, _NANCH_PAD), 1)
    in_range = col < _NANCH

    def _cnt(pred):
        return jnp.sum(jnp.where(pred & in_range, 1, 0), axis=1,
                       keepdims=True)

    t = jnp.where(_cnt(key >= 0) >= _PRE,
                  jnp.zeros((_NUM_IMAGES, 1), jnp.int32),
                  jnp.full((_NUM_IMAGES, 1), jnp.int32(-2147483648)))
    for b in range(30, -1, -1):
        t_try = t + jnp.int32(1 << b)
        t = jnp.where(_cnt(key >= t_try) >= _PRE, t_try, t)

    need = _PRE - _cnt(key > t)
    lo = jnp.zeros((_NUM_IMAGES, 1), jnp.int32)
    hi = jnp.full((_NUM_IMAGES, 1), jnp.int32(_NANCH - 1))
    for _ in range(15):
        mid = (lo + hi) // 2
        ok = _cnt((key == t) & (col <= mid)) >= need
        hi = jnp.where(ok, mid, hi)
        lo = jnp.where(ok, lo, mid + 1)

    sel = (key > t) | ((key == t) & (col <= hi))
    sel_ref[...] = jnp.where(sel & in_range, 1, 0).astype(jnp.int32)


def _nms_body(bx_ref, sc_ref, masked_ref, boxes_ref, d_ref):
    x1 = jnp.clip(bx_ref[0], 0.0, _IMG_W)
    y1 = jnp.clip(bx_ref[1], 0.0, _IMG_H)
    x2 = jnp.clip(bx_ref[2], 0.0, _IMG_W)
    y2 = jnp.clip(bx_ref[3], 0.0, _IMG_H)
    scores = jax.nn.sigmoid(sc_ref[...])
    ws = x2 - x1
    hs = y2 - y1
    area = ws * hs
    valid = (ws >= _MIN_SIZE) & (hs >= _MIN_SIZE) & (scores > _SCORE_THRESH)
    keep = jnp.where(valid, 1.0, 0.0)

    col = jax.lax.broadcasted_iota(jnp.int32, (1, _PAD), 1)
    lane = jax.lax.broadcasted_iota(jnp.int32, (1, _B), 1)

    x1b, y1b, x2b, y2b = (a[:, None, :] for a in (x1, y1, x2, y2))
    areab = area[:, None, :]

    for bi in range(_NBLK):
        s = bi * _B
        rem = _PAD - s
        bx1 = x1b[:, :, s:s + _B].reshape(_NUM_IMAGES, _B, 1)
        by1 = y1b[:, :, s:s + _B].reshape(_NUM_IMAGES, _B, 1)
        bx2 = x2b[:, :, s:s + _B].reshape(_NUM_IMAGES, _B, 1)
        by2 = y2b[:, :, s:s + _B].reshape(_NUM_IMAGES, _B, 1)
        barea = areab[:, :, s:s + _B].reshape(_NUM_IMAGES, _B, 1)
        xs1, ys1, xs2, ys2 = (a[:, :, s:] for a in (x1b, y1b, x2b, y2b))
        areas = areab[:, :, s:]
        iw = jnp.clip(jnp.minimum(bx2, xs2) - jnp.maximum(bx1, xs1), 0.0, None)
        ih = jnp.clip(jnp.minimum(by2, ys2) - jnp.maximum(by1, ys1), 0.0, None)
        inter = iw * ih
        union = barea + areas - inter
        iou = inter / jnp.maximum(union, 1e-9)
        m = jnp.where(iou > _NMS_THRESH, 1.0, 0.0)

        d_ref[...] = m[:, :, :_B]
        kblk = keep[:, s:s + _B]

        def _step(t, kb):
            row = d_ref[:, pl.ds(t, 1), :].reshape(_NUM_IMAGES, _B)
            kt = jnp.sum(jnp.where(lane == t, kb, 0.0), axis=1, keepdims=True)
            supp = (row > 0.0) & (lane > t) & (kt > 0.0)
            return jnp.where(supp, 0.0, kb)

        kblk = jax.lax.fori_loop(0, _B, _step, kblk, unroll=8)

        if s + _B < _PAD:
            supp_any = jnp.max(m[:, :, _B:] * kblk[:, :, None], axis=1)
            tail = jnp.where(supp_any > 0.0, 0.0, keep[:, s + _B:])
            parts = [keep[:, :s], kblk, tail]
        else:
            parts = [keep[:, :s], kblk]
        parts = [p for p in parts if p.shape[1] > 0]
        keep = jnp.concatenate(parts, axis=1) if len(parts) > 1 else parts[0]

    masked_ref[...] = jnp.where(keep > 0.0, scores, -1.0)
    boxes_ref[0] = x1
    boxes_ref[1] = y1
    boxes_ref[2] = x2
    boxes_ref[3] = y2


@jax.jit
def kernel(proposals, objectness):
    objectness = jax.lax.stop_gradient(objectness)
    obj_pad = jnp.pad(objectness, ((0, 0), (0, _NANCH_PAD - _NANCH)),
                      constant_values=-jnp.inf)
    sel = pl.pallas_call(
        _cutoff_body,
        out_shape=jax.ShapeDtypeStruct((_NUM_IMAGES, _NANCH_PAD), jnp.int32),
    )(obj_pad)[:, :_NANCH]

    pos = jnp.cumsum(sel, axis=1) - 1
    pos = jnp.where(sel > 0, pos, _PAD)
    rows = jnp.broadcast_to(jnp.arange(_NUM_IMAGES)[:, None], pos.shape)
    comp_scores = jnp.full((_NUM_IMAGES, _PAD), -jnp.inf, jnp.float32)
    comp_scores = comp_scores.at[rows, pos].set(objectness, mode="drop")
    comp_idx = jnp.zeros((_NUM_IMAGES, _PAD), jnp.int32)
    aidx = jnp.broadcast_to(jnp.arange(_NANCH, dtype=jnp.int32)[None],
                            pos.shape)
    comp_idx = comp_idx.at[rows, pos].set(aidx, mode="drop")

    top_scores, cpos = jax.lax.top_k(comp_scores, _PRE)
    top_idx = jnp.take_along_axis(comp_idx, cpos, axis=1)
    boxes = jnp.take_along_axis(proposals, top_idx[..., None], axis=1)

    bx = jnp.transpose(boxes, (2, 0, 1))
    bx = jnp.pad(bx, ((0, 0), (0, 0), (0, _PAD - _PRE)))
    sc = jnp.pad(top_scores, ((0, 0), (0, _PAD - _PRE)),
                 constant_values=-1e30)

    masked, cboxes = pl.pallas_call(
        _nms_body,
        out_shape=[
            jax.ShapeDtypeStruct((_NUM_IMAGES, _PAD), jnp.float32),
            jax.ShapeDtypeStruct((4, _NUM_IMAGES, _PAD), jnp.float32),
        ],
        scratch_shapes=[pltpu.VMEM((_NUM_IMAGES, _B, _B), jnp.float32)],
    )(bx, sc)

    masked = masked[:, :_PRE]
    cboxes = jnp.transpose(cboxes[:, :, :_PRE], (1, 2, 0))
    final_scores, kidx = jax.lax.top_k(masked, _POST)
    final_boxes = jnp.take_along_axis(cboxes, kidx[..., None], axis=1)
    return jnp.concatenate([final_boxes, final_scores[..., None]], axis=-1)


# speedup vs baseline: 2.3424x; 2.3424x over previous
import jax
import jax.numpy as jnp
from jax.experimental import pallas as pl
from jax.experimental.pallas import tpu as pltpu

_NUM_IMAGES = 4
_PRE = 2000
_PAD = 2048
_POST = 1000
_B = 128
_NBLK = _PAD // _B
_IMG_H = 800.0
_IMG_W = 800.0
_NMS_THRESH = 0.7
_SCORE_THRESH = 0.0
_MIN_SIZE = 0.001


def _nms_body(bx_ref, sc_ref, masked_ref, boxes_ref, d_ref):
    x1 = jnp.clip(bx_ref[0], 0.0, _IMG_W)
    y1 = jnp.clip(bx_ref[1], 0.0, _IMG_H)
    x2 = jnp.clip(bx_ref[2], 0.0, _IMG_W)
    y2 = jnp.clip(bx_ref[3], 0.0, _IMG_H)
    scores = jax.nn.sigmoid(sc_ref[...])
    ws = x2 - x1
    hs = y2 - y1
    area = ws * hs
    valid = (ws >= _MIN_SIZE) & (hs >= _MIN_SIZE) & (scores > _SCORE_THRESH)
    keep = jnp.where(valid, 1.0, 0.0)

    col = jax.lax.broadcasted_iota(jnp.int32, (1, _PAD), 1)
    lane = jax.lax.broadcasted_iota(jnp.int32, (1, _B), 1)

    x1b, y1b, x2b, y2b = (a[:, None, :] for a in (x1, y1, x2, y2))
    areab = area[:, None, :]

    r_iota = jax.lax.broadcasted_iota(jnp.int32, (_B, 1, _B), 0)
    c_iota = jax.lax.broadcasted_iota(jnp.int32, (_B, 1, _B), 2)
    dmask = c_iota > r_iota

    for bi in range(_NBLK):
        s = bi * _B
        rem = _PAD - s
        bx1 = x1[:, s:s + _B].T[:, :, None]
        by1 = y1[:, s:s + _B].T[:, :, None]
        bx2 = x2[:, s:s + _B].T[:, :, None]
        by2 = y2[:, s:s + _B].T[:, :, None]
        barea = area[:, s:s + _B].T[:, :, None]
        xs1, ys1, xs2, ys2 = (a[None, :, s:] for a in (x1, y1, x2, y2))
        areas = area[None, :, s:]
        iw = jnp.clip(jnp.minimum(bx2, xs2) - jnp.maximum(bx1, xs1), 0.0, None)
        ih = jnp.clip(jnp.minimum(by2, ys2) - jnp.maximum(by1, ys1), 0.0, None)
        inter = iw * ih
        union = barea + areas - inter
        iou = inter / jnp.maximum(union, 1e-9)
        m = jnp.where(iou > _NMS_THRESH, 1.0, 0.0)

        d_ref[...] = jnp.where(dmask, m[:, :, :_B], 0.0)
        kblk = keep[:, s:s + _B]

        def _step(t, kb):
            row = d_ref[pl.ds(t, 1)].reshape(_NUM_IMAGES, _B)
            kt = jnp.sum(jnp.where(lane == t, kb, 0.0), axis=1, keepdims=True)
            supp = (row > 0.0) & (kt > 0.0)
            return jnp.where(supp, 0.0, kb)

        kblk = jax.lax.fori_loop(0, _B, _step, kblk, unroll=8)

        if s + _B < _PAD:
            supp_any = jnp.max(m[:, :, _B:] * kblk.T[:, :, None], axis=0)
            tail = jnp.where(supp_any > 0.0, 0.0, keep[:, s + _B:])
            parts = [keep[:, :s], kblk, tail]
        else:
            parts = [keep[:, :s], kblk]
        parts = [p for p in parts if p.shape[1] > 0]
        keep = jnp.concatenate(parts, axis=1) if len(parts) > 1 else parts[0]

    masked_ref[...] = jnp.where(keep > 0.0, scores, -1.0)
    boxes_ref[0] = x1
    boxes_ref[1] = y1
    boxes_ref[2] = x2
    boxes_ref[3] = y2


@jax.jit
def kernel(proposals, objectness):
    objectness = jax.lax.stop_gradient(objectness)
    top_scores, top_idx = jax.lax.top_k(objectness, _PRE)
    boxes = jnp.take_along_axis(proposals, top_idx[..., None], axis=1)

    bx = jnp.transpose(boxes, (2, 0, 1))
    bx = jnp.pad(bx, ((0, 0), (0, 0), (0, _PAD - _PRE)))
    sc = jnp.pad(top_scores, ((0, 0), (0, _PAD - _PRE)),
                 constant_values=-1e30)

    masked, cboxes = pl.pallas_call(
        _nms_body,
        out_shape=[
            jax.ShapeDtypeStruct((_NUM_IMAGES, _PAD), jnp.float32),
            jax.ShapeDtypeStruct((4, _NUM_IMAGES, _PAD), jnp.float32),
        ],
        scratch_shapes=[pltpu.VMEM((_B, _NUM_IMAGES, _B), jnp.float32)],
    )(bx, sc)

    masked = masked[:, :_PRE]
    cboxes = jnp.transpose(cboxes[:, :, :_PRE], (1, 2, 0))
    final_scores, kidx = jax.lax.top_k(masked, _POST)
    final_boxes = jnp.take_along_axis(cboxes, kidx[..., None], axis=1)
    return jnp.concatenate([final_boxes, final_scores[..., None]], axis=-1)
